# baseline (speedup 1.0000x reference)
import jax
import jax.numpy as jnp
from jax.experimental import pallas as pl

N_CLS = 1000
PROMPT_LEN = 16
EMBED_DIM = 768
CTX_LEN = 77
SUF_LEN = CTX_LEN - 1 - PROMPT_LEN


def _concat_body(pre_ref, prm_ref, suf_ref, out_ref):
    out_ref[...] = jnp.zeros_like(out_ref)


def kernel(new_prompt_tokens, token_prefix, token_suffix, tokenized_prompts):
    C = 40
    prompts = pl.pallas_call(
        _concat_body,
        grid=(N_CLS // C,),
        in_specs=[
            pl.BlockSpec((C, 1, EMBED_DIM), lambda i: (i, 0, 0)),
            pl.BlockSpec((C, PROMPT_LEN, EMBED_DIM), lambda i: (i, 0, 0)),
            pl.BlockSpec((C, SUF_LEN, EMBED_DIM), lambda i: (i, 0, 0)),
        ],
        out_specs=pl.BlockSpec((C, CTX_LEN, EMBED_DIM), lambda i: (i, 0, 0)),
        out_shape=jax.ShapeDtypeStruct((N_CLS, CTX_LEN, EMBED_DIM), jnp.float32),
    )(token_prefix, new_prompt_tokens, token_suffix)
    return (tokenized_prompts, prompts)


# speedup vs baseline: 1.0009x; 1.0009x over previous
import jax
import jax.numpy as jnp
from jax import lax
from jax.experimental import pallas as pl
from jax.experimental.pallas import tpu as pltpu

N_CLS = 1000
PROMPT_LEN = 16
EMBED_DIM = 768
CTX_LEN = 77
SUF_LEN = CTX_LEN - 1 - PROMPT_LEN

C = 8
NSTEP = N_CLS // C
NBUF = 8
LA = 6


def _body(pre_hbm, prm_hbm, suf_hbm, out_hbm,
          pre_v, prm_v, suf_v, out_v,
          pre_s, prm_s, suf_s, out_s):
    i = pl.program_id(0)

    def in_copies(step):
        slot = lax.rem(step, NBUF)
        c0 = step * C
        return (
            pltpu.make_async_copy(pre_hbm.at[pl.ds(c0, C)], pre_v.at[slot], pre_s.at[slot]),
            pltpu.make_async_copy(prm_hbm.at[pl.ds(c0, C)], prm_v.at[slot], prm_s.at[slot]),
            pltpu.make_async_copy(suf_hbm.at[pl.ds(c0, C)], suf_v.at[slot], suf_s.at[slot]),
        )

    def out_copy(step):
        slot = lax.rem(step, NBUF)
        c0 = step * C
        return pltpu.make_async_copy(out_v.at[slot], out_hbm.at[pl.ds(c0, C)], out_s.at[slot])

    @pl.when(i == 0)
    def _prologue():
        for j in range(LA):
            for cp in in_copies(j):
                cp.start()

    slot = lax.rem(i, NBUF)
    for cp in in_copies(i):
        cp.wait()

    @pl.when(i >= NBUF)
    def _wait_prev_out():
        out_copy(i - NBUF).wait()

    out_v[slot] = jnp.concatenate(
        [pre_v[slot], prm_v[slot], suf_v[slot]], axis=1)
    out_copy(i).start()

    @pl.when(i + LA < NSTEP)
    def _next_in():
        for cp in in_copies(i + LA):
            cp.start()

    @pl.when(i == NSTEP - 1)
    def _drain():
        for j in range(NBUF):
            out_copy(NSTEP - 1 - j).wait()


def kernel(new_prompt_tokens, token_prefix, token_suffix, tokenized_prompts):
    prompts = pl.pallas_call(
        _body,
        grid=(NSTEP,),
        in_specs=[
            pl.BlockSpec(memory_space=pl.ANY),
            pl.BlockSpec(memory_space=pl.ANY),
            pl.BlockSpec(memory_space=pl.ANY),
        ],
        out_specs=pl.BlockSpec(memory_space=pl.ANY),
        out_shape=jax.ShapeDtypeStruct((N_CLS, CTX_LEN, EMBED_DIM), jnp.float32),
        scratch_shapes=[
            pltpu.VMEM((NBUF, C, 1, EMBED_DIM), jnp.float32),
            pltpu.VMEM((NBUF, C, PROMPT_LEN, EMBED_DIM), jnp.float32),
            pltpu.VMEM((NBUF, C, SUF_LEN, EMBED_DIM), jnp.float32),
            pltpu.VMEM((NBUF, C, CTX_LEN, EMBED_DIM), jnp.float32),
            pltpu.SemaphoreType.DMA((NBUF,)),
            pltpu.SemaphoreType.DMA((NBUF,)),
            pltpu.SemaphoreType.DMA((NBUF,)),
            pltpu.SemaphoreType.DMA((NBUF,)),
        ],
        compiler_params=pltpu.CompilerParams(
            dimension_semantics=("arbitrary",),
        ),
    )(token_prefix, new_prompt_tokens, token_suffix)
    return (tokenized_prompts, prompts)
